# baseline (speedup 1.0000x reference)
import functools

import jax
import jax.numpy as jnp
from jax import lax
from jax.experimental import pallas as pl
from jax.experimental.pallas import tpu as pltpu
from jax.experimental.pallas import tpu_sc as plsc


@functools.cache
def _build(B, V, D):
    info = plsc.get_sparse_core_info()
    nw = info.num_cores * info.num_subcores
    b_per_w = B // nw
    assert B % (8 * nw) == 0

    mesh = plsc.VectorSubcoreMesh(core_axis_name="c", subcore_axis_name="s")

    @functools.partial(
        pl.kernel,
        mesh=mesh,
        compiler_params=pltpu.CompilerParams(use_tc_tiling_on_sc=True),
        out_type=jax.ShapeDtypeStruct((B, D), jnp.float32),
        scratch_types=[
            pltpu.VMEM((b_per_w,), jnp.int32),
            pltpu.VMEM((b_per_w, D), jnp.float32),
            pltpu.SemaphoreType.DMA,
            pltpu.SemaphoreType.DMA,
            pltpu.SemaphoreType.DMA,
        ],
    )
    def k(idx_hbm, table_hbm, out_hbm, idx_v, rows_v, sem0, sem1, osem):
        L = info.num_lanes
        half = b_per_w // 2
        wid = lax.axis_index("s") * info.num_cores + lax.axis_index("c")
        base = wid * b_per_w
        pltpu.sync_copy(idx_hbm.at[pl.ds(base, b_per_w)], idx_v)

        def make_fire(sem):
            def fire(c, _):
                vec = idx_v[pl.ds(c * L, L)]
                for j in range(L):
                    row = vec[j]
                    pltpu.async_copy(table_hbm.at[pl.ds(row, 1)],
                                     rows_v.at[pl.ds(c * L + j, 1)], sem)
                return 0
            return fire

        nch = b_per_w // L
        lax.fori_loop(0, nch // 2, make_fire(sem0), 0)
        lax.fori_loop(nch // 2, nch, make_fire(sem1), 0)
        pltpu.make_async_copy(table_hbm.at[pl.ds(0, half)],
                              rows_v.at[pl.ds(0, half)], sem0).wait()
        st0 = pltpu.async_copy(rows_v.at[pl.ds(0, half)],
                               out_hbm.at[pl.ds(base, half)], osem)
        pltpu.make_async_copy(table_hbm.at[pl.ds(0, half)],
                              rows_v.at[pl.ds(half, half)], sem1).wait()
        st1 = pltpu.async_copy(rows_v.at[pl.ds(half, half)],
                               out_hbm.at[pl.ds(base + half, half)], osem)
        st0.wait()
        st1.wait()

    return k


def kernel(spk_ids, embed_weight):
    B, = spk_ids.shape
    V, D = embed_weight.shape
    return _build(B, V, D)(spk_ids.astype(jnp.int32), embed_weight)


# speedup vs baseline: 1.0193x; 1.0193x over previous
import functools

import jax
import jax.numpy as jnp
from jax import lax
from jax.experimental import pallas as pl
from jax.experimental.pallas import tpu as pltpu
from jax.experimental.pallas import tpu_sc as plsc


@functools.cache
def _build(B, V, D):
    info = plsc.get_sparse_core_info()
    nw = info.num_cores * info.num_subcores
    b_per_w = B // nw
    assert B % (8 * nw) == 0

    mesh = plsc.VectorSubcoreMesh(core_axis_name="c", subcore_axis_name="s")

    @functools.partial(
        pl.kernel,
        mesh=mesh,
        compiler_params=pltpu.CompilerParams(use_tc_tiling_on_sc=True),
        out_type=jax.ShapeDtypeStruct((B, D), jnp.float32),
        scratch_types=[
            pltpu.VMEM((b_per_w,), jnp.int32),
            pltpu.VMEM((b_per_w, D), jnp.float32),
            pltpu.SemaphoreType.DMA,
        ],
    )
    def k(idx_hbm, table_hbm, out_hbm, idx_v, rows_v, sem):
        L = info.num_lanes
        wid = lax.axis_index("s") * info.num_cores + lax.axis_index("c")
        base = wid * b_per_w
        pltpu.sync_copy(idx_hbm.at[pl.ds(base, b_per_w)], idx_v)

        def fire(c, _):
            vec = idx_v[pl.ds(c * L, L)]
            for j in range(L):
                row = vec[j]
                pltpu.async_copy(table_hbm.at[pl.ds(row, 1)],
                                 rows_v.at[pl.ds(c * L + j, 1)], sem)
            return 0

        lax.fori_loop(0, b_per_w // L, fire, 0)
        pltpu.make_async_copy(table_hbm.at[pl.ds(0, b_per_w)], rows_v,
                              sem).wait()
        pltpu.sync_copy(rows_v, out_hbm.at[pl.ds(base, b_per_w)])

    return k


def kernel(spk_ids, embed_weight):
    B, = spk_ids.shape
    V, D = embed_weight.shape
    return _build(B, V, D)(spk_ids.astype(jnp.int32), embed_weight)
